# baseline (speedup 1.0000x reference)
import functools

import jax
import jax.numpy as jnp
from jax import lax
from jax.experimental import pallas as pl
from jax.experimental.pallas import tpu as pltpu
from jax.experimental.pallas import tpu_sc as plsc

NW = 32
SLAB = 31104
CW = 384
NPIECE = 16
RG = 128


def kernel(x, token_table, pos_table):
    B, S = x.shape
    V, D = token_table.shape
    N = B * S
    vmain = (V // 128) * 128
    ntail = V - vmain
    last_n = vmain - SLAB * (NW - 1)
    tt_T = token_table.T
    piece = N // NPIECE
    xp = x.reshape(NPIECE, piece).astype(jnp.int32)
    mesh = plsc.VectorSubcoreMesh(core_axis_name="c", subcore_axis_name="s")

    @functools.partial(
        pl.kernel,
        mesh=mesh,
        out_type=jax.ShapeDtypeStruct((N + 128, 128), jnp.float32),
        scratch_types=[
            pltpu.VMEM((piece,), jnp.int32),
            pltpu.VMEM((N,), jnp.int32),
            pltpu.VMEM((RG * 16,), jnp.int32),
            pltpu.VMEM((2, D, CW), jnp.float32),
            pltpu.VMEM((2, 128, 128), jnp.float32),
            pltpu.VMEM((2, 128), jnp.int32),
            pltpu.SemaphoreType.DMA,
            pltpu.SemaphoreType.DMA,
        ],
        compiler_params=pltpu.CompilerParams(
            needs_layout_passes=False, use_tc_tiling_on_sc=True),
    )
    def run(x_hbm, tok_hbm, out_hbm,
            xs_v, hit_v, loc_v, slab_v, row_v, pos_v, gsem, osem):
        cid = lax.axis_index("c")
        sid = lax.axis_index("s")
        wid = sid * 2 + cid
        is_last = wid == NW - 1
        lo = wid * SLAB
        hi = jnp.where(is_last, vmain, lo + SLAB)
        nch = jnp.where(is_last, last_n // CW, SLAB // CW)
        lanes = lax.iota(jnp.int32, 16)

        def piece_loop(pi, pv):
            pltpu.sync_copy(x_hbm.at[pi], xs_v)

            def group_loop(g, pv2):
                xv = xs_v[pl.ds(g * 16, 16)]
                m = (xv >= lo) & (xv < hi)
                mi = m.astype(jnp.int32)
                packed = ((xv - lo) << 16) | (pi * piece + g * 16 + lanes)
                plsc.store_scatter(
                    hit_v, [pv2 + plsc.cumsum(mi) - 1], packed, mask=m)
                return pv2 + plsc.all_reduce_population_count(m)

            return lax.fori_loop(0, piece // 16, group_loop, pv)

        pv = lax.fori_loop(0, NPIECE, piece_loop, jnp.zeros((16,), jnp.int32))
        nhits = jnp.max(pv)
        nh_groups = (nhits + 15) // 16
        nrounds = (nh_groups + RG - 1) // RG

        def hbm_chunk(c):
            return tok_hbm.at[:, pl.ds(lo + c * CW, CW)]

        pltpu.async_copy(hbm_chunk(0), slab_v.at[0], gsem)

        def chunk_loop(c, carry):
            buf = lax.rem(c, 2)
            pltpu.make_async_copy(hbm_chunk(0), slab_v.at[0], gsem).wait()
            pltpu.async_copy(
                hbm_chunk(jnp.minimum(c + 1, nch - 1)),
                slab_v.at[lax.rem(c + 1, 2)], gsem)
            c0 = c * CW

            def round_loop(r, carry2):
                def scan_group(k, qv2):
                    g = r * RG + k
                    pk = hit_v[pl.ds(g * 16, 16)]
                    vl = (pk >> 16) & 0xFFFF
                    live = (g * 16 + lanes) < nhits
                    m = live & (vl >= c0) & (vl < c0 + CW)
                    mi = m.astype(jnp.int32)
                    plsc.store_scatter(
                        loc_v, [qv2 + plsc.cumsum(mi) - 1],
                        pk - (c0 << 16), mask=m)
                    return qv2 + plsc.all_reduce_population_count(m)

                qv = lax.fori_loop(
                    0, RG, scan_group, jnp.zeros((16,), jnp.int32))
                q = jnp.max(qv)

                def emit_group(e, nr):
                    nf2 = nr // 128
                    fb = lax.rem(nf2, 2)
                    slot = lax.rem(nr, 128)
                    pk = loc_v[pl.ds(e * 16, 16)]
                    live = (e * 16 + lanes) < q
                    vl = jnp.where(live, (pk >> 16) & 0xFFFF, 0)
                    pp = jnp.where(live, pk & 0xFFFF, N)
                    bv = jnp.full((16,), 0, jnp.int32) + buf
                    fbv = jnp.full((16,), 0, jnp.int32) + fb

                    @pl.when((slot == 0) & (nf2 >= 2))
                    def _():
                        pltpu.make_async_copy(
                            row_v.at[0], out_hbm.at[pl.ds(N, 128)],
                            osem).wait()

                    for d in range(D):
                        dv = jnp.full((16,), 0, jnp.int32) + d
                        w = plsc.load_gather(slab_v, [bv, dv, vl])
                        plsc.store_scatter(row_v, [fbv, slot + lanes, dv], w)
                    pos_v[fb, pl.ds(slot, 16)] = pp

                    @pl.when(slot == 112)
                    def _():
                        pltpu.async_copy(
                            row_v.at[fb], out_hbm.at[pos_v.at[fb]], osem)

                    return nr + 16

                return lax.fori_loop(0, (q + 15) // 16, emit_group, carry2)

            return lax.fori_loop(0, nrounds, round_loop, carry)

        nr = lax.fori_loop(0, nch, chunk_loop, 0)
        pltpu.make_async_copy(hbm_chunk(0), slab_v.at[0], gsem).wait()

        nf2 = nr // 128
        slot = lax.rem(nr, 128)
        fb = lax.rem(nf2, 2)

        @pl.when(slot > 0)
        def _():
            def pad(k, c2):
                @pl.when(k * 16 >= slot)
                def _():
                    pos_v[fb, pl.ds(k * 16, 16)] = jnp.full((16,), N, jnp.int32)
                return c2

            lax.fori_loop(0, 8, pad, 0)

            pltpu.async_copy(
                row_v.at[fb], out_hbm.at[pos_v.at[fb]], osem)

        nflush = nf2 + jnp.where(slot > 0, 1, 0)

        def drain(i, c2):
            pltpu.make_async_copy(
                row_v.at[0], out_hbm.at[pl.ds(N, 128)], osem).wait()
            return c2

        lax.fori_loop(0, jnp.minimum(nflush, 2), drain, 0)

    out = run(xp, tt_T)[:N, :D].reshape(B, S, D)
    tail_tab = token_table[vmain:]
    onehot = (jnp.clip(x - vmain, -1, ntail - 1)[..., None]
              == jnp.arange(ntail)[None, None, :]).astype(jnp.float32)
    fix = jnp.einsum("bsv,vd->bsd", onehot, tail_tab)
    out = jnp.where((x >= vmain)[..., None], fix, out)
    return out + pos_table[None, :, :]


# speedup vs baseline: 2.3997x; 2.3997x over previous
import functools

import jax
import jax.numpy as jnp
from jax import lax
from jax.experimental import pallas as pl
from jax.experimental.pallas import tpu as pltpu
from jax.experimental.pallas import tpu_sc as plsc

NW = 32
CH = 128


def kernel(x, token_table, pos_table):
    B, S = x.shape
    V, D = token_table.shape
    N = B * S
    per_w = N // NW
    nch = per_w // CH
    xf = x.reshape(NW, nch, CH).astype(jnp.int32)
    mesh = plsc.VectorSubcoreMesh(core_axis_name="c", subcore_axis_name="s")

    @functools.partial(
        pl.kernel,
        mesh=mesh,
        out_type=jax.ShapeDtypeStruct((N, D), jnp.float32),
        scratch_types=[
            pltpu.VMEM((nch, CH), jnp.int32),
            pltpu.VMEM((CH, D), jnp.float32),
            pltpu.VMEM((CH, D), jnp.float32),
            pltpu.SemaphoreType.DMA,
            pltpu.SemaphoreType.DMA,
        ],
        compiler_params=pltpu.CompilerParams(
            needs_layout_passes=False, use_tc_tiling_on_sc=True),
    )
    def run(x_hbm, tok_hbm, pos_hbm, out_hbm, idx_v, tok_v, out_v, gsem, psem):
        cid = lax.axis_index("c")
        sid = lax.axis_index("s")
        wid = sid * 2 + cid
        base = wid * per_w
        s_base = lax.rem(base, S)
        pltpu.sync_copy(x_hbm.at[wid], idx_v)
        lanes = lax.iota(jnp.int32, 16)

        def chunk(c, carry):
            p = pltpu.async_copy(
                pos_hbm.at[pl.ds(s_base + c * CH, CH)], out_v, psem)
            for g in range(CH // 16):
                xv = idx_v[c, pl.ds(g * 16, 16)]
                for l in range(16):
                    v = jnp.max(jnp.where(lanes == l, xv, 0))
                    pltpu.async_copy(
                        tok_hbm.at[pl.ds(v, 1)],
                        tok_v.at[pl.ds(g * 16 + l, 1)], gsem)
            p.wait()
            drain = pltpu.make_async_copy(
                tok_hbm.at[pl.ds(0, 1)], tok_v.at[pl.ds(0, 1)], gsem)
            for r in range(CH):
                drain.wait()
            for r in range(CH):
                for j in range(D // 16):
                    sl = pl.ds(j * 16, 16)
                    out_v[r, sl] = out_v[r, sl] + tok_v[r, sl]
            pltpu.sync_copy(out_v, out_hbm.at[pl.ds(base + c * CH, CH)])
            return carry

        lax.fori_loop(0, nch, chunk, 0)

    out = run(xf, token_table, pos_table)
    return out.reshape(B, S, D)
